# baseline (speedup 1.0000x reference)
import functools

import jax
import jax.numpy as jnp
from jax import lax
from jax.experimental import pallas as pl
from jax.experimental.pallas import tpu as pltpu
from jax.experimental.pallas import tpu_sc as plsc

N = 2048
DIM = 128
NC = 2
K = 512
HD = DIM // NC
BN = 512

NT = N * NC
NW = 32
BPW = NT // NW
L = 16


def _select_kernel(x_ref, embed_ref, i1_ref, i2_ref):
    x = x_ref[...]
    lane_iota = jax.lax.broadcasted_iota(jnp.int32, (BN, K), 1)
    i1_cols = []
    i2_cols = []
    for c in range(NC):
        xc = x[:, c * HD:(c + 1) * HD]
        ec = embed_ref[c]
        ecT = jnp.transpose(ec)
        s = jax.lax.dot_general(
            xc, ecT, (((1,), (0,)), ((), ())),
            preferred_element_type=jnp.float32,
            precision=jax.lax.Precision.HIGHEST)
        en = jnp.sum(ecT * ecT, axis=0, keepdims=True)
        d = en - 2.0 * s
        m1 = jnp.min(d, axis=1, keepdims=True)
        i1 = jnp.min(jnp.where(d == m1, lane_iota, K), axis=1,
                     keepdims=True)
        d2m = jnp.where(lane_iota == i1, jnp.inf, d)
        m2 = jnp.min(d2m, axis=1, keepdims=True)
        i2 = jnp.min(jnp.where(d2m == m2, lane_iota, K), axis=1,
                     keepdims=True)
        i1_cols.append(i1)
        i2_cols.append(i2)
    i1_ref[...] = jnp.concatenate(i1_cols, axis=1)
    i2_ref[...] = jnp.concatenate(i2_cols, axis=1)


def _refine_kernel(xf_hbm, ef_hbm, i1_hbm, i2_hbm, qf_hbm, ind_hbm,
                   i1_v, i2_v, fi1_v, fi2_v, x_v, r1_v, r2_v, q_v, ind_v,
                   sem1, sem2):
    wid = lax.axis_index("s") * 2 + lax.axis_index("c")
    base = wid * BPW
    pltpu.sync_copy(i1_hbm.at[pl.ds(base, BPW)], i1_v)
    pltpu.sync_copy(i2_hbm.at[pl.ds(base, BPW)], i2_v)
    pltpu.sync_copy(xf_hbm.at[pl.ds(base, BPW)], x_v)
    off = (lax.iota(jnp.int32, L) % 2) * K
    for j in range(BPW // L):
        fi1_v[pl.ds(j * L, L)] = i1_v[pl.ds(j * L, L)] + off
        fi2_v[pl.ds(j * L, L)] = i2_v[pl.ds(j * L, L)] + off
    cp1 = pltpu.make_async_copy(ef_hbm.at[fi1_v], r1_v, sem1)
    cp2 = pltpu.make_async_copy(ef_hbm.at[fi2_v], r2_v, sem2)
    cp1.start()
    cp2.start()
    cp1.wait()
    cp2.wait()

    lane = lax.iota(jnp.int32, L)
    _dnums = lax.GatherDimensionNumbers(
        offset_dims=(), collapsed_slice_dims=(0,), start_index_map=(0,))

    def _shuffle(v, idx):
        return lax.gather(
            v, idx.reshape(L, 1), _dnums, (1,),
            mode=lax.GatherScatterMode.PROMISE_IN_BOUNDS)

    def _allsum(v):
        for sh in (8, 4, 2, 1):
            v = v + _shuffle(v, lane ^ sh)
        return v

    def group_body(g, carry):
        j1v = i1_v[pl.ds(g * L, L)]
        j2v = i2_v[pl.ds(g * L, L)]
        indv = jnp.zeros((L,), jnp.int32)
        for t in range(L):
            i = g * L + t
            acc1 = jnp.zeros((L,), jnp.float32)
            acc2 = jnp.zeros((L,), jnp.float32)
            r1k = []
            r2k = []
            for k in range(HD // L):
                xc = x_v[i, pl.ds(k * L, L)]
                r1 = r1_v[i, pl.ds(k * L, L)]
                r2 = r2_v[i, pl.ds(k * L, L)]
                r1k.append(r1)
                r2k.append(r2)
                a = xc - r1
                acc1 = acc1 + a * a
                b = xc - r2
                acc2 = acc2 + b * b
            d1 = _allsum(acc1)
            d2 = _allsum(acc2)
            tsplat = jnp.full((L,), t, jnp.int32)
            j1 = _shuffle(j1v, tsplat)
            j2 = _shuffle(j2v, tsplat)
            lt = jnp.where(d2 < d1, 1, 0)
            eq = jnp.where(d2 == d1, 1, 0)
            jl = jnp.where(j2 < j1, 1, 0)
            take2 = (lt + eq * jl) > 0
            indv = jnp.where(lane == t, jnp.where(take2, j2, j1), indv)
            for k in range(HD // L):
                q_v[i, pl.ds(k * L, L)] = jnp.where(take2, r2k[k], r1k[k])
        ind_v[pl.ds(g * L, L)] = indv
        return carry

    lax.fori_loop(0, BPW // L, group_body, 0)
    pltpu.sync_copy(q_v, qf_hbm.at[pl.ds(base, BPW)])
    pltpu.sync_copy(ind_v, ind_hbm.at[pl.ds(base, BPW)])


@jax.jit
def kernel(x, node_type, embed):
    del node_type
    i1, i2 = pl.pallas_call(
        _select_kernel,
        grid=(N // BN,),
        in_specs=[
            pl.BlockSpec((BN, DIM), lambda i: (i, 0)),
            pl.BlockSpec((NC, K, HD), lambda i: (0, 0, 0)),
        ],
        out_specs=[
            pl.BlockSpec((BN, NC), lambda i: (i, 0)),
            pl.BlockSpec((BN, NC), lambda i: (i, 0)),
        ],
        out_shape=[
            jax.ShapeDtypeStruct((N, NC), jnp.int32),
            jax.ShapeDtypeStruct((N, NC), jnp.int32),
        ],
    )(x, embed)

    refine = functools.partial(
        pl.kernel,
        out_type=[
            jax.ShapeDtypeStruct((NT, HD), jnp.float32),
            jax.ShapeDtypeStruct((NT,), jnp.int32),
        ],
        mesh=plsc.VectorSubcoreMesh(core_axis_name="c", subcore_axis_name="s"),
        scratch_types=[
            pltpu.VMEM((BPW,), jnp.int32),
            pltpu.VMEM((BPW,), jnp.int32),
            pltpu.VMEM((BPW,), jnp.int32),
            pltpu.VMEM((BPW,), jnp.int32),
            pltpu.VMEM((BPW, HD), jnp.float32),
            pltpu.VMEM((BPW, 2 * HD), jnp.float32),
            pltpu.VMEM((BPW, 2 * HD), jnp.float32),
            pltpu.VMEM((BPW, HD), jnp.float32),
            pltpu.VMEM((BPW,), jnp.int32),
            pltpu.SemaphoreType.DMA,
            pltpu.SemaphoreType.DMA,
        ],
    )(_refine_kernel)

    ef_padded = jnp.pad(embed.reshape(NC * K, HD), ((0, 0), (0, HD)))
    qf, indf = refine(
        x.reshape(NT, HD),
        ef_padded,
        i1.reshape(NT),
        i2.reshape(NT),
    )
    return (qf.reshape(N, DIM), indf.reshape(N, NC), 0)


# speedup vs baseline: 2.0493x; 2.0493x over previous
import jax
import jax.numpy as jnp
import numpy as np
from jax.experimental import pallas as pl

N = 2048
DIM = 128
NC = 2
K = 512
HD = DIM // NC
BN = 512

_HI_MASK = np.uint32(0xFFFF0000)


def _bf16_split3(v):
    hi_f = jax.lax.bitcast_convert_type(
        jax.lax.bitcast_convert_type(v, jnp.uint32) & _HI_MASK, jnp.float32)
    r1 = v - hi_f
    mid_f = jax.lax.bitcast_convert_type(
        jax.lax.bitcast_convert_type(r1, jnp.uint32) & _HI_MASK, jnp.float32)
    lo_f = r1 - mid_f
    return (hi_f.astype(jnp.bfloat16), mid_f.astype(jnp.bfloat16),
            lo_f.astype(jnp.bfloat16))


def _gather_rows(oh, ec3):
    parts = [
        jax.lax.dot_general(
            oh, t, (((1,), (0,)), ((), ())),
            preferred_element_type=jnp.float32)
        for t in ec3
    ]
    return (parts[0] + parts[1]) + parts[2]


def _vq_kernel(x_ref, embed_ref, q_ref, idx_ref):
    x = x_ref[...]
    lane_iota = jax.lax.broadcasted_iota(jnp.int32, (BN, K), 1)

    xcs = []
    ecs = []
    ds = []
    for c in range(NC):
        xc = x[:, c * HD:(c + 1) * HD]
        ec = embed_ref[c]
        ecT = jnp.transpose(ec)
        s = jax.lax.dot_general(
            xc, ecT, (((1,), (0,)), ((), ())),
            preferred_element_type=jnp.float32,
            precision=jax.lax.Precision.HIGHEST)
        en = jnp.sum(ecT * ecT, axis=0, keepdims=True)
        xcs.append(xc)
        ecs.append(ec)
        ds.append(en - 2.0 * s)

    i1s = []
    i2s = []
    for c in range(NC):
        d = ds[c]
        m1 = jnp.min(d, axis=1, keepdims=True)
        i1 = jnp.min(jnp.where(d == m1, lane_iota, K), axis=1,
                     keepdims=True)
        d2m = jnp.where(lane_iota == i1, jnp.inf, d)
        m2 = jnp.min(d2m, axis=1, keepdims=True)
        i2 = jnp.min(jnp.where(d2m == m2, lane_iota, K), axis=1,
                     keepdims=True)
        i1s.append(i1)
        i2s.append(i2)

    idx_cols = []
    q_cols = []
    for c in range(NC):
        ec3 = _bf16_split3(ecs[c])
        oh1 = (lane_iota == i1s[c]).astype(jnp.bfloat16)
        oh2 = (lane_iota == i2s[c]).astype(jnp.bfloat16)
        e1 = _gather_rows(oh1, ec3)
        e2 = _gather_rows(oh2, ec3)
        r1 = xcs[c] - e1
        r2 = xcs[c] - e2
        d1 = jnp.sum(r1 * r1, axis=1, keepdims=True)
        d2 = jnp.sum(r2 * r2, axis=1, keepdims=True)
        take2 = (d2 < d1) | ((d2 == d1) & (i2s[c] < i1s[c]))
        idx_cols.append(jnp.where(take2, i2s[c], i1s[c]))
        q_cols.append(jnp.where(take2, e2, e1))
    q_ref[...] = jnp.concatenate(q_cols, axis=1)
    idx_ref[...] = jnp.concatenate(idx_cols, axis=1)


@jax.jit
def kernel(x, node_type, embed):
    del node_type
    grid = (N // BN,)
    q, idx = pl.pallas_call(
        _vq_kernel,
        grid=grid,
        in_specs=[
            pl.BlockSpec((BN, DIM), lambda i: (i, 0)),
            pl.BlockSpec((NC, K, HD), lambda i: (0, 0, 0)),
        ],
        out_specs=[
            pl.BlockSpec((BN, DIM), lambda i: (i, 0)),
            pl.BlockSpec((BN, NC), lambda i: (i, 0)),
        ],
        out_shape=[
            jax.ShapeDtypeStruct((N, DIM), jnp.float32),
            jax.ShapeDtypeStruct((N, NC), jnp.int32),
        ],
    )(x, embed)
    return (q, idx, 0)


# speedup vs baseline: 2.0742x; 1.0122x over previous
import jax
import jax.numpy as jnp
import numpy as np
from jax.experimental import pallas as pl

N = 2048
DIM = 128
NC = 2
K = 512
HD = DIM // NC
BN = 512

_HI_MASK = np.uint32(0xFFFF0000)


def _bf16_split3(v):
    hi_f = jax.lax.bitcast_convert_type(
        jax.lax.bitcast_convert_type(v, jnp.uint32) & _HI_MASK, jnp.float32)
    r1 = v - hi_f
    mid_f = jax.lax.bitcast_convert_type(
        jax.lax.bitcast_convert_type(r1, jnp.uint32) & _HI_MASK, jnp.float32)
    lo_f = r1 - mid_f
    return (hi_f.astype(jnp.bfloat16), mid_f.astype(jnp.bfloat16),
            lo_f.astype(jnp.bfloat16))


def _gather_rows(oh, ec3):
    parts = [
        jax.lax.dot_general(
            oh, t, (((1,), (0,)), ((), ())),
            preferred_element_type=jnp.float32)
        for t in ec3
    ]
    return (parts[0] + parts[1]) + parts[2]


def _vq_kernel(x_ref, embed_ref, q_ref, idx_ref):
    x = x_ref[...]
    lane_f = jax.lax.broadcasted_iota(
        jnp.int32, (BN, K), 1).astype(jnp.float32)
    lane_f2 = jax.lax.broadcasted_iota(
        jnp.int32, (2 * BN, K), 1).astype(jnp.float32)
    kf = jnp.float32(K)

    xcs = []
    ecs = []
    ds = []
    for c in range(NC):
        xc = x[:, c * HD:(c + 1) * HD]
        ec = embed_ref[c]
        ecT = jnp.transpose(ec)
        s = jax.lax.dot_general(
            xc, ecT, (((1,), (0,)), ((), ())),
            preferred_element_type=jnp.float32,
            precision=jax.lax.Precision.HIGHEST)
        en = jnp.sum(ecT * ecT, axis=0, keepdims=True)
        xcs.append(xc)
        ecs.append(ec)
        ds.append(en - 2.0 * s)

    i1s = []
    i2s = []
    for c in range(NC):
        d = ds[c]
        m1 = jnp.min(d, axis=1, keepdims=True)
        i1 = jnp.min(jnp.where(d == m1, lane_f, kf), axis=1,
                     keepdims=True)
        d2m = jnp.where(lane_f == i1, jnp.inf, d)
        m2 = jnp.min(d2m, axis=1, keepdims=True)
        i2 = jnp.min(jnp.where(d2m == m2, lane_f, kf), axis=1,
                     keepdims=True)
        i1s.append(i1)
        i2s.append(i2)

    idx_cols = []
    q_cols = []
    for c in range(NC):
        ec3 = _bf16_split3(ecs[c])
        i12 = jnp.concatenate([i1s[c], i2s[c]], axis=0)
        oh12 = (lane_f2 == i12).astype(jnp.bfloat16)
        e12 = _gather_rows(oh12, ec3)
        e1 = e12[:BN]
        e2 = e12[BN:]
        r1 = xcs[c] - e1
        r2 = xcs[c] - e2
        d1 = jnp.sum(r1 * r1, axis=1, keepdims=True)
        d2 = jnp.sum(r2 * r2, axis=1, keepdims=True)
        take2 = (d2 < d1) | ((d2 == d1) & (i2s[c] < i1s[c]))
        idx_cols.append(jnp.where(take2, i2s[c], i1s[c]).astype(jnp.int32))
        q_cols.append(jnp.where(take2, e2, e1))
    q_ref[...] = jnp.concatenate(q_cols, axis=1)
    idx_ref[...] = jnp.concatenate(idx_cols, axis=1)


@jax.jit
def kernel(x, node_type, embed):
    del node_type
    grid = (N // BN,)
    q, idx = pl.pallas_call(
        _vq_kernel,
        grid=grid,
        in_specs=[
            pl.BlockSpec((BN, DIM), lambda i: (i, 0)),
            pl.BlockSpec((NC, K, HD), lambda i: (0, 0, 0)),
        ],
        out_specs=[
            pl.BlockSpec((BN, DIM), lambda i: (i, 0)),
            pl.BlockSpec((BN, NC), lambda i: (i, 0)),
        ],
        out_shape=[
            jax.ShapeDtypeStruct((N, DIM), jnp.float32),
            jax.ShapeDtypeStruct((N, NC), jnp.int32),
        ],
    )(x, embed)
    return (q, idx, 0)


# speedup vs baseline: 2.1767x; 1.0494x over previous
import jax
import jax.numpy as jnp
import numpy as np
from jax.experimental import pallas as pl

N = 2048
DIM = 128
NC = 2
K = 512
HD = DIM // NC
BN = 1024

_HI_MASK = np.uint32(0xFFFF0000)


def _bf16_split3(v):
    hi_f = jax.lax.bitcast_convert_type(
        jax.lax.bitcast_convert_type(v, jnp.uint32) & _HI_MASK, jnp.float32)
    r1 = v - hi_f
    mid_f = jax.lax.bitcast_convert_type(
        jax.lax.bitcast_convert_type(r1, jnp.uint32) & _HI_MASK, jnp.float32)
    lo_f = r1 - mid_f
    return (hi_f.astype(jnp.bfloat16), mid_f.astype(jnp.bfloat16),
            lo_f.astype(jnp.bfloat16))


def _gather_rows(oh, ec3):
    parts = [
        jax.lax.dot_general(
            oh, t, (((1,), (0,)), ((), ())),
            preferred_element_type=jnp.float32)
        for t in ec3
    ]
    return (parts[0] + parts[1]) + parts[2]


def _vq_kernel(x_ref, embed_ref, q_ref, idx_ref):
    x = x_ref[...]
    lane_f = jax.lax.broadcasted_iota(
        jnp.int32, (BN, K), 1).astype(jnp.float32)
    lane_f2 = jax.lax.broadcasted_iota(
        jnp.int32, (2 * BN, K), 1).astype(jnp.float32)
    kf = jnp.float32(K)

    xcs = []
    ecs = []
    ds = []
    for c in range(NC):
        xc = x[:, c * HD:(c + 1) * HD]
        ec = embed_ref[c]
        ecT = jnp.transpose(ec)
        s = jax.lax.dot_general(
            xc, ecT, (((1,), (0,)), ((), ())),
            preferred_element_type=jnp.float32,
            precision=jax.lax.Precision.HIGHEST)
        en = jnp.sum(ecT * ecT, axis=0, keepdims=True)
        xcs.append(xc)
        ecs.append(ec)
        ds.append(en - 2.0 * s)

    i1s = []
    i2s = []
    for c in range(NC):
        d = ds[c]
        m1 = jnp.min(d, axis=1, keepdims=True)
        i1 = jnp.min(jnp.where(d == m1, lane_f, kf), axis=1,
                     keepdims=True)
        d2m = jnp.where(lane_f == i1, jnp.inf, d)
        m2 = jnp.min(d2m, axis=1, keepdims=True)
        i2 = jnp.min(jnp.where(d2m == m2, lane_f, kf), axis=1,
                     keepdims=True)
        i1s.append(i1)
        i2s.append(i2)

    idx_cols = []
    q_cols = []
    for c in range(NC):
        ec3 = _bf16_split3(ecs[c])
        i12 = jnp.concatenate([i1s[c], i2s[c]], axis=0)
        oh12 = (lane_f2 == i12).astype(jnp.bfloat16)
        e12 = _gather_rows(oh12, ec3)
        e1 = e12[:BN]
        e2 = e12[BN:]
        r1 = xcs[c] - e1
        r2 = xcs[c] - e2
        d1 = jnp.sum(r1 * r1, axis=1, keepdims=True)
        d2 = jnp.sum(r2 * r2, axis=1, keepdims=True)
        take2 = (d2 < d1) | ((d2 == d1) & (i2s[c] < i1s[c]))
        idx_cols.append(jnp.where(take2, i2s[c], i1s[c]).astype(jnp.int32))
        q_cols.append(jnp.where(take2, e2, e1))
    q_ref[...] = jnp.concatenate(q_cols, axis=1)
    idx_ref[...] = jnp.concatenate(idx_cols, axis=1)


@jax.jit
def kernel(x, node_type, embed):
    del node_type
    grid = (N // BN,)
    q, idx = pl.pallas_call(
        _vq_kernel,
        grid=grid,
        in_specs=[
            pl.BlockSpec((BN, DIM), lambda i: (i, 0)),
            pl.BlockSpec((NC, K, HD), lambda i: (0, 0, 0)),
        ],
        out_specs=[
            pl.BlockSpec((BN, DIM), lambda i: (i, 0)),
            pl.BlockSpec((BN, NC), lambda i: (i, 0)),
        ],
        out_shape=[
            jax.ShapeDtypeStruct((N, DIM), jnp.float32),
            jax.ShapeDtypeStruct((N, NC), jnp.int32),
        ],
    )(x, embed)
    return (q, idx, 0)
